# Initial kernel scaffold; baseline (speedup 1.0000x reference)
import jax
import jax.numpy as jnp
from jax.experimental import pallas as pl


def kernel(feat, edge_index, weight1, weight2, bias):
    raise NotImplementedError("write your pallas kernel here")


# speedup vs baseline: 13.2193x; 13.2193x over previous
import functools

import jax
import jax.numpy as jnp
from jax import lax
from jax.experimental import pallas as pl
from jax.experimental.pallas import tpu as pltpu
from jax.experimental.pallas import tpu_sc as plsc

N = 10000
E = 160000
D = 256
H = 128
NC = 2
NS = 16
NP = 10240
RPT = NP // NS

C1 = 40
NCH1 = (E // (NC * NS)) // C1
C3 = 80
NCH3 = (E // NS) // C3

_mesh = plsc.VectorSubcoreMesh(core_axis_name="c", subcore_axis_name="s")


@functools.partial(
    pl.kernel,
    out_type=jax.ShapeDtypeStruct((NC, NP, H), jnp.float32),
    mesh=_mesh,
    scratch_types=[
        pltpu.VMEM((NCH1, C1), jnp.int32),
        pltpu.VMEM((C1, H), jnp.float32),
        pltpu.VMEM_SHARED((NP, H), jnp.float32),
    ],
)
def _deg_kernel(src_hbm, ones_hbm, zrows_hbm, out_hbm, idx_v, ones_v, deg_sh):
    c = lax.axis_index("c")
    s = lax.axis_index("s")
    pltpu.sync_copy(src_hbm.at[c, s], idx_v)
    pltpu.sync_copy(ones_hbm, ones_v)
    pltpu.sync_copy(zrows_hbm, deg_sh.at[pl.ds(s * RPT, RPT)])
    plsc.subcore_barrier()

    def body(j, carry):
        pltpu.sync_copy(ones_v, deg_sh.at[idx_v.at[j]], add=True)
        return carry

    lax.fori_loop(0, NCH1, body, 0)
    plsc.subcore_barrier()
    pltpu.sync_copy(deg_sh.at[pl.ds(s * RPT, RPT)],
                    out_hbm.at[c, pl.ds(s * RPT, RPT)])


@functools.partial(
    pl.kernel,
    out_type=jax.ShapeDtypeStruct((NC, NP, H), jnp.float32),
    mesh=_mesh,
    scratch_types=[
        pltpu.VMEM((NCH3, C3), jnp.int32),
        pltpu.VMEM((NCH3, C3), jnp.int32),
        pltpu.VMEM((C3, H), jnp.float32),
        pltpu.SemaphoreType.DMA,
        pltpu.VMEM_SHARED((NP, H), jnp.float32),
    ],
)
def _scatter_kernel(g2_hbm, src_hbm, dst_hbm, zrows_hbm, out_hbm,
                    idx_s, idx_d, rows_v, sem, acc_sh):
    c = lax.axis_index("c")
    s = lax.axis_index("s")
    pltpu.sync_copy(src_hbm.at[c, s], idx_s)
    pltpu.sync_copy(dst_hbm.at[s], idx_d)
    pltpu.sync_copy(zrows_hbm, acc_sh.at[pl.ds(s * RPT, RPT)])
    plsc.subcore_barrier()

    def body(j, carry):
        pltpu.async_copy(g2_hbm.at[idx_s.at[j]], rows_v, sem).wait()
        pltpu.sync_copy(rows_v, acc_sh.at[idx_d.at[j]], add=True)
        return carry

    lax.fori_loop(0, NCH3, body, 0)
    plsc.subcore_barrier()
    pltpu.sync_copy(acc_sh.at[pl.ds(s * RPT, RPT)],
                    out_hbm.at[c, pl.ds(s * RPT, RPT)])


_RB = 1000
_GRID = N // _RB


def _scale_body(deg_ref, feat_ref, out_ref):
    d = deg_ref[0, :, 0:1] + deg_ref[1, :, 0:1]
    r = lax.rsqrt(jnp.maximum(d, 1.0))
    g = feat_ref[...] * r
    out_ref[0] = g[:, :H]
    out_ref[1] = g[:, H:]


def _tail_body(deg_ref, feat_ref, a_ref, w1_ref, w2_ref, b_ref, out_ref):
    d = deg_ref[0, :, 0:1] + deg_ref[1, :, 0:1]
    r = lax.rsqrt(jnp.maximum(d, 1.0))
    f = feat_ref[...]
    a = jnp.concatenate([a_ref[0], a_ref[1]], axis=1)
    b1 = f + r * a
    b2 = (f * a) * r
    h = (jnp.dot(b1, w1_ref[...], preferred_element_type=jnp.float32)
         + jnp.dot(b2, w2_ref[...], preferred_element_type=jnp.float32)
         + b_ref[...])
    h = jnp.where(h >= 0.0, h, 0.01 * h)
    nrm = jnp.sqrt(jnp.sum(h * h, axis=1, keepdims=True))
    out_ref[...] = h / jnp.maximum(nrm, 1e-12)


def kernel(feat, edge_index, weight1, weight2, bias):
    src = edge_index[0]
    dst = edge_index[1]
    src1 = src.reshape(NC, NS, NCH1, C1)
    srcr = src.reshape(NS, NCH3, C3)
    src2 = jnp.stack([srcr, srcr + N])
    dstr = dst.reshape(NS, NCH3, C3)
    ones1 = jnp.ones((C1, H), jnp.float32)
    z128 = jnp.zeros((RPT, H), jnp.float32)

    deg_parts = _deg_kernel(src1, ones1, z128)

    g2 = pl.pallas_call(
        _scale_body,
        grid=(_GRID,),
        in_specs=[
            pl.BlockSpec((2, _RB, H), lambda i: (0, i, 0)),
            pl.BlockSpec((_RB, D), lambda i: (i, 0)),
        ],
        out_specs=pl.BlockSpec((2, _RB, H), lambda i: (0, i, 0)),
        out_shape=jax.ShapeDtypeStruct((2, N, H), jnp.float32),
    )(deg_parts, feat)
    g2 = g2.reshape(2 * N, H)

    a_parts = _scatter_kernel(g2, src2, dstr, z128)

    out = pl.pallas_call(
        _tail_body,
        grid=(_GRID,),
        in_specs=[
            pl.BlockSpec((2, _RB, H), lambda i: (0, i, 0)),
            pl.BlockSpec((_RB, D), lambda i: (i, 0)),
            pl.BlockSpec((2, _RB, H), lambda i: (0, i, 0)),
            pl.BlockSpec((D, D), lambda i: (0, 0)),
            pl.BlockSpec((D, D), lambda i: (0, 0)),
            pl.BlockSpec((1, D), lambda i: (0, 0)),
        ],
        out_specs=pl.BlockSpec((_RB, D), lambda i: (i, 0)),
        out_shape=jax.ShapeDtypeStruct((N, D), jnp.float32),
    )(deg_parts, feat, a_parts, weight1, weight2, bias.reshape(1, D))
    return out


# speedup vs baseline: 16.1252x; 1.2198x over previous
import functools

import jax
import jax.numpy as jnp
from jax import lax
from jax.experimental import pallas as pl
from jax.experimental.pallas import tpu as pltpu
from jax.experimental.pallas import tpu_sc as plsc

N = 10000
E = 160000
D = 256
H = 128
NC = 2
NS = 16
NP = 10240
RPT = NP // NS

C1 = 40
NCH1 = (E // (NC * NS)) // C1
C3 = 80
NCH3 = (E // NS) // C3

_mesh = plsc.VectorSubcoreMesh(core_axis_name="c", subcore_axis_name="s")


@functools.partial(
    pl.kernel,
    out_type=jax.ShapeDtypeStruct((NC, NP, H), jnp.float32),
    mesh=_mesh,
    scratch_types=[
        pltpu.VMEM((NCH1, C1), jnp.int32),
        pltpu.VMEM((C1, H), jnp.float32),
        pltpu.VMEM_SHARED((NP, H), jnp.float32),
    ],
)
def _deg_kernel(src_hbm, ones_hbm, zrows_hbm, out_hbm, idx_v, ones_v, deg_sh):
    c = lax.axis_index("c")
    s = lax.axis_index("s")
    pltpu.sync_copy(src_hbm.at[c, s], idx_v)
    pltpu.sync_copy(ones_hbm, ones_v)
    pltpu.sync_copy(zrows_hbm, deg_sh.at[pl.ds(s * RPT, RPT)])
    plsc.subcore_barrier()

    def body(j, carry):
        pltpu.sync_copy(ones_v, deg_sh.at[idx_v.at[j]], add=True)
        return carry

    lax.fori_loop(0, NCH1, body, 0)
    plsc.subcore_barrier()
    pltpu.sync_copy(deg_sh.at[pl.ds(s * RPT, RPT)],
                    out_hbm.at[c, pl.ds(s * RPT, RPT)])


@functools.partial(
    pl.kernel,
    out_type=jax.ShapeDtypeStruct((NC, NP, H), jnp.float32),
    mesh=_mesh,
    scratch_types=[
        pltpu.VMEM((2, C3), jnp.int32),
        pltpu.VMEM((2, C3), jnp.int32),
        pltpu.VMEM((C3, H), jnp.float32),
        pltpu.VMEM((C3, H), jnp.float32),
        pltpu.SemaphoreType.DMA,
        pltpu.SemaphoreType.DMA,
        pltpu.SemaphoreType.DMA,
        pltpu.SemaphoreType.DMA,
        pltpu.VMEM_SHARED((NP, H), jnp.float32),
    ],
)
def _scatter_kernel(g2_hbm, eidx_hbm, zrows_hbm, out_hbm,
                    ib0, ib1, rows0, rows1, semi0, semi1, sem0, sem1, acc_sh):
    c = lax.axis_index("c")
    s = lax.axis_index("s")
    pltpu.sync_copy(zrows_hbm, acc_sh.at[pl.ds(s * RPT, RPT)])
    plsc.subcore_barrier()

    pltpu.async_copy(eidx_hbm.at[c, s, 0], ib0, semi0)
    pltpu.async_copy(eidx_hbm.at[c, s, 1], ib1, semi1)
    pltpu.make_async_copy(eidx_hbm.at[c, s, 0], ib0, semi0).wait()
    pltpu.async_copy(g2_hbm.at[ib0.at[0]], rows0, sem0)
    pltpu.make_async_copy(eidx_hbm.at[c, s, 1], ib1, semi1).wait()
    pltpu.async_copy(g2_hbm.at[ib1.at[0]], rows1, sem1)

    def pair(k, carry):
        j0 = 2 * k
        j1 = 2 * k + 1
        pltpu.make_async_copy(g2_hbm.at[ib0.at[0]], rows0, sem0).wait()
        pltpu.sync_copy(rows0, acc_sh.at[ib0.at[1]], add=True)
        pltpu.async_copy(eidx_hbm.at[c, s, j0 + 2], ib0, semi0)
        pltpu.make_async_copy(g2_hbm.at[ib1.at[0]], rows1, sem1).wait()
        pltpu.sync_copy(rows1, acc_sh.at[ib1.at[1]], add=True)
        pltpu.async_copy(eidx_hbm.at[c, s, j1 + 2], ib1, semi1)
        pltpu.make_async_copy(eidx_hbm.at[c, s, j0 + 2], ib0, semi0).wait()
        pltpu.async_copy(g2_hbm.at[ib0.at[0]], rows0, sem0)
        pltpu.make_async_copy(eidx_hbm.at[c, s, j1 + 2], ib1, semi1).wait()
        pltpu.async_copy(g2_hbm.at[ib1.at[0]], rows1, sem1)
        return carry

    lax.fori_loop(0, (NCH3 - 1) // 2 - 1, pair, 0)
    pltpu.make_async_copy(g2_hbm.at[ib0.at[0]], rows0, sem0).wait()
    pltpu.sync_copy(rows0, acc_sh.at[ib0.at[1]], add=True)
    pltpu.async_copy(eidx_hbm.at[c, s, NCH3 - 1], ib0, semi0)
    pltpu.make_async_copy(g2_hbm.at[ib1.at[0]], rows1, sem1).wait()
    pltpu.sync_copy(rows1, acc_sh.at[ib1.at[1]], add=True)
    pltpu.make_async_copy(eidx_hbm.at[c, s, NCH3 - 1], ib0, semi0).wait()
    pltpu.async_copy(g2_hbm.at[ib0.at[0]], rows0, sem0)
    pltpu.make_async_copy(g2_hbm.at[ib0.at[0]], rows0, sem0).wait()
    pltpu.sync_copy(rows0, acc_sh.at[ib0.at[1]], add=True)
    plsc.subcore_barrier()
    pltpu.sync_copy(acc_sh.at[pl.ds(s * RPT, RPT)],
                    out_hbm.at[c, pl.ds(s * RPT, RPT)])


_RB = 1000
_GRID = N // _RB


def _scale_body(deg_ref, feat_ref, out_ref):
    d = deg_ref[0, :, 0:1] + deg_ref[1, :, 0:1]
    r = lax.rsqrt(jnp.maximum(d, 1.0))
    g = feat_ref[...] * r
    out_ref[0] = g[:, :H]
    out_ref[1] = g[:, H:]


def _tail_body(deg_ref, feat_ref, a_ref, w1_ref, w2_ref, b_ref, out_ref):
    d = deg_ref[0, :, 0:1] + deg_ref[1, :, 0:1]
    r = lax.rsqrt(jnp.maximum(d, 1.0))
    f = feat_ref[...]
    a = jnp.concatenate([a_ref[0], a_ref[1]], axis=1)
    b1 = f + r * a
    b2 = (f * a) * r
    h = (jnp.dot(b1, w1_ref[...], preferred_element_type=jnp.float32)
         + jnp.dot(b2, w2_ref[...], preferred_element_type=jnp.float32)
         + b_ref[...])
    h = jnp.where(h >= 0.0, h, 0.01 * h)
    nrm = jnp.sqrt(jnp.sum(h * h, axis=1, keepdims=True))
    out_ref[...] = h / jnp.maximum(nrm, 1e-12)


def kernel(feat, edge_index, weight1, weight2, bias):
    src = edge_index[0]
    dst = edge_index[1]
    src1 = src.reshape(NC, NS, NCH1, C1)
    srcr = src.reshape(NS, NCH3, C3)
    dstr = dst.reshape(NS, NCH3, C3)
    eidx = jnp.stack([jnp.stack([srcr, dstr], axis=2),
                      jnp.stack([srcr + N, dstr], axis=2)])
    ones1 = jnp.ones((C1, H), jnp.float32)
    z128 = jnp.zeros((RPT, H), jnp.float32)

    deg_parts = _deg_kernel(src1, ones1, z128)

    g2 = pl.pallas_call(
        _scale_body,
        grid=(_GRID,),
        in_specs=[
            pl.BlockSpec((2, _RB, H), lambda i: (0, i, 0)),
            pl.BlockSpec((_RB, D), lambda i: (i, 0)),
        ],
        out_specs=pl.BlockSpec((2, _RB, H), lambda i: (0, i, 0)),
        out_shape=jax.ShapeDtypeStruct((2, N, H), jnp.float32),
    )(deg_parts, feat)
    g2 = g2.reshape(2 * N, H)

    a_parts = _scatter_kernel(g2, eidx, z128)

    out = pl.pallas_call(
        _tail_body,
        grid=(_GRID,),
        in_specs=[
            pl.BlockSpec((2, _RB, H), lambda i: (0, i, 0)),
            pl.BlockSpec((_RB, D), lambda i: (i, 0)),
            pl.BlockSpec((2, _RB, H), lambda i: (0, i, 0)),
            pl.BlockSpec((D, D), lambda i: (0, 0)),
            pl.BlockSpec((D, D), lambda i: (0, 0)),
            pl.BlockSpec((1, D), lambda i: (0, 0)),
        ],
        out_specs=pl.BlockSpec((_RB, D), lambda i: (i, 0)),
        out_shape=jax.ShapeDtypeStruct((N, D), jnp.float32),
    )(deg_parts, feat, a_parts, weight1, weight2, bias.reshape(1, D))
    return out


# speedup vs baseline: 16.2706x; 1.0090x over previous
import functools

import jax
import jax.numpy as jnp
from jax import lax
from jax.experimental import pallas as pl
from jax.experimental.pallas import tpu as pltpu
from jax.experimental.pallas import tpu_sc as plsc

N = 10000
E = 160000
D = 256
H = 128
NC = 2
NS = 16
NP = 10240
RPT = NP // NS

C1 = 40
NCH1 = (E // (NC * NS)) // C1
C3 = 80
NCH3 = (E // NS) // C3

_mesh = plsc.VectorSubcoreMesh(core_axis_name="c", subcore_axis_name="s")


@functools.partial(
    pl.kernel,
    out_type=jax.ShapeDtypeStruct((NC, NP, H), jnp.float32),
    mesh=_mesh,
    scratch_types=[
        pltpu.VMEM((NCH1, C1), jnp.int32),
        pltpu.VMEM((C1, H), jnp.float32),
        pltpu.VMEM_SHARED((NP, H), jnp.float32),
    ],
)
def _deg_kernel(src_hbm, ones_hbm, zrows_hbm, out_hbm, idx_v, ones_v, deg_sh):
    c = lax.axis_index("c")
    s = lax.axis_index("s")
    pltpu.sync_copy(src_hbm.at[c, s], idx_v)
    pltpu.sync_copy(ones_hbm, ones_v)
    pltpu.sync_copy(zrows_hbm, deg_sh.at[pl.ds(s * RPT, RPT)])
    plsc.subcore_barrier()

    def body(j, carry):
        pltpu.sync_copy(ones_v, deg_sh.at[idx_v.at[j]], add=True)
        return carry

    lax.fori_loop(0, NCH1, body, 0)
    plsc.subcore_barrier()
    pltpu.sync_copy(deg_sh.at[pl.ds(s * RPT, RPT)],
                    out_hbm.at[c, pl.ds(s * RPT, RPT)])


@functools.partial(
    pl.kernel,
    out_type=jax.ShapeDtypeStruct((NC, NP, H), jnp.float32),
    mesh=_mesh,
    scratch_types=[
        pltpu.VMEM((2, C3), jnp.int32),
        pltpu.VMEM((2, C3), jnp.int32),
        pltpu.VMEM((C3, H), jnp.float32),
        pltpu.VMEM((C3, H), jnp.float32),
        pltpu.SemaphoreType.DMA,
        pltpu.SemaphoreType.DMA,
        pltpu.SemaphoreType.DMA,
        pltpu.SemaphoreType.DMA,
        pltpu.VMEM_SHARED((NP, H), jnp.float32),
    ],
)
def _scatter_kernel(g2_hbm, eidx_hbm, zrows_hbm, out_hbm,
                    ib0, ib1, rows0, rows1, semi0, semi1, sem0, sem1, acc_sh):
    c = lax.axis_index("c")
    s = lax.axis_index("s")
    pltpu.sync_copy(zrows_hbm, acc_sh.at[pl.ds(s * RPT, RPT)])
    plsc.subcore_barrier()

    pltpu.async_copy(eidx_hbm.at[c, s, 0], ib0, semi0)
    pltpu.async_copy(eidx_hbm.at[c, s, 1], ib1, semi1)
    pltpu.make_async_copy(eidx_hbm.at[c, s, 0], ib0, semi0).wait()
    pltpu.async_copy(g2_hbm.at[ib0.at[0]], rows0, sem0)
    pltpu.make_async_copy(eidx_hbm.at[c, s, 1], ib1, semi1).wait()
    pltpu.async_copy(g2_hbm.at[ib1.at[0]], rows1, sem1)

    def pair(k, carry):
        j0 = 2 * k
        j1 = 2 * k + 1
        pltpu.make_async_copy(g2_hbm.at[ib0.at[0]], rows0, sem0).wait()
        pltpu.sync_copy(rows0, acc_sh.at[ib0.at[1]], add=True)
        pltpu.async_copy(eidx_hbm.at[c, s, j0 + 2], ib0, semi0)
        pltpu.make_async_copy(g2_hbm.at[ib1.at[0]], rows1, sem1).wait()
        pltpu.sync_copy(rows1, acc_sh.at[ib1.at[1]], add=True)
        pltpu.async_copy(eidx_hbm.at[c, s, j1 + 2], ib1, semi1)
        pltpu.make_async_copy(eidx_hbm.at[c, s, j0 + 2], ib0, semi0).wait()
        pltpu.async_copy(g2_hbm.at[ib0.at[0]], rows0, sem0)
        pltpu.make_async_copy(eidx_hbm.at[c, s, j1 + 2], ib1, semi1).wait()
        pltpu.async_copy(g2_hbm.at[ib1.at[0]], rows1, sem1)
        return carry

    lax.fori_loop(0, (NCH3 - 1) // 2 - 1, pair, 0)
    pltpu.make_async_copy(g2_hbm.at[ib0.at[0]], rows0, sem0).wait()
    pltpu.sync_copy(rows0, acc_sh.at[ib0.at[1]], add=True)
    pltpu.async_copy(eidx_hbm.at[c, s, NCH3 - 1], ib0, semi0)
    pltpu.make_async_copy(g2_hbm.at[ib1.at[0]], rows1, sem1).wait()
    pltpu.sync_copy(rows1, acc_sh.at[ib1.at[1]], add=True)
    pltpu.make_async_copy(eidx_hbm.at[c, s, NCH3 - 1], ib0, semi0).wait()
    pltpu.async_copy(g2_hbm.at[ib0.at[0]], rows0, sem0)
    pltpu.make_async_copy(g2_hbm.at[ib0.at[0]], rows0, sem0).wait()
    pltpu.sync_copy(rows0, acc_sh.at[ib0.at[1]], add=True)
    plsc.subcore_barrier()
    pltpu.sync_copy(acc_sh.at[pl.ds(s * RPT, RPT)],
                    out_hbm.at[c, pl.ds(s * RPT, RPT)])


_RB = 2000
_GRID = N // _RB


def _scale_body(deg_ref, feat_ref, out_ref):
    d = deg_ref[0, :, 0:1] + deg_ref[1, :, 0:1]
    r = lax.rsqrt(jnp.maximum(d, 1.0))
    g = feat_ref[...] * r
    out_ref[0] = g[:, :H]
    out_ref[1] = g[:, H:]


def _tail_body(deg_ref, feat_ref, a_ref, w1_ref, w2_ref, b_ref, out_ref):
    d = deg_ref[0, :, 0:1] + deg_ref[1, :, 0:1]
    r = lax.rsqrt(jnp.maximum(d, 1.0))
    f = feat_ref[...]
    a = jnp.concatenate([a_ref[0], a_ref[1]], axis=1)
    b1 = f + r * a
    b2 = (f * a) * r
    h = (jnp.dot(b1, w1_ref[...], preferred_element_type=jnp.float32)
         + jnp.dot(b2, w2_ref[...], preferred_element_type=jnp.float32)
         + b_ref[...])
    h = jnp.where(h >= 0.0, h, 0.01 * h)
    nrm = jnp.sqrt(jnp.sum(h * h, axis=1, keepdims=True))
    out_ref[...] = h / jnp.maximum(nrm, 1e-12)


def kernel(feat, edge_index, weight1, weight2, bias):
    src = edge_index[0]
    dst = edge_index[1]
    src1 = src.reshape(NC, NS, NCH1, C1)
    srcr = src.reshape(NS, NCH3, C3)
    dstr = dst.reshape(NS, NCH3, C3)
    eidx = jnp.stack([jnp.stack([srcr, dstr], axis=2),
                      jnp.stack([srcr + N, dstr], axis=2)])
    ones1 = jnp.ones((C1, H), jnp.float32)
    z128 = jnp.zeros((RPT, H), jnp.float32)

    deg_parts = _deg_kernel(src1, ones1, z128)

    g2 = pl.pallas_call(
        _scale_body,
        grid=(_GRID,),
        in_specs=[
            pl.BlockSpec((2, _RB, H), lambda i: (0, i, 0)),
            pl.BlockSpec((_RB, D), lambda i: (i, 0)),
        ],
        out_specs=pl.BlockSpec((2, _RB, H), lambda i: (0, i, 0)),
        out_shape=jax.ShapeDtypeStruct((2, N, H), jnp.float32),
    )(deg_parts, feat)
    g2 = g2.reshape(2 * N, H)

    a_parts = _scatter_kernel(g2, eidx, z128)

    out = pl.pallas_call(
        _tail_body,
        grid=(_GRID,),
        in_specs=[
            pl.BlockSpec((2, _RB, H), lambda i: (0, i, 0)),
            pl.BlockSpec((_RB, D), lambda i: (i, 0)),
            pl.BlockSpec((2, _RB, H), lambda i: (0, i, 0)),
            pl.BlockSpec((D, D), lambda i: (0, 0)),
            pl.BlockSpec((D, D), lambda i: (0, 0)),
            pl.BlockSpec((1, D), lambda i: (0, 0)),
        ],
        out_specs=pl.BlockSpec((_RB, D), lambda i: (i, 0)),
        out_shape=jax.ShapeDtypeStruct((N, D), jnp.float32),
    )(deg_parts, feat, a_parts, weight1, weight2, bias.reshape(1, D))
    return out
